# initial kernel scaffold (unmeasured)
import jax
import jax.numpy as jnp
from jax import lax
from jax.experimental import pallas as pl
from jax.experimental.pallas import tpu as pltpu


def kernel(
    x,
):
    def body(*refs):
        pass

    out_shape = jax.ShapeDtypeStruct(..., jnp.float32)
    return pl.pallas_call(body, out_shape=out_shape)(...)


# baseline (device time: 70059 ns/iter reference)
import jax
import jax.numpy as jnp
from jax import lax
from jax.experimental import pallas as pl
from jax.experimental.pallas import tpu as pltpu

W = 32


def kernel(x):
    m, n = x.shape
    mc = m // W

    def body(x_ref, out_ref, rs_ref, send1, recv1, send2, recv2):
        my = lax.axis_index("i")

        rs_ref[pl.ds(my * mc, mc), :] = x_ref[pl.ds(my * mc, mc), :]

        sends1 = []
        for j in range(W):
            rdma = pltpu.make_async_remote_copy(
                src_ref=x_ref.at[pl.ds(j * mc, mc), :],
                dst_ref=rs_ref.at[pl.ds(my * mc, mc), :],
                send_sem=send1.at[j],
                recv_sem=recv1.at[my],
                device_id=(j,),
                device_id_type=pl.DeviceIdType.MESH,
            )
            sends1.append(rdma)

        for j in range(W):
            @pl.when(my != j)
            def _(rdma=sends1[j]):
                rdma.start()

        for i in range(W):
            recv = pltpu.make_async_remote_copy(
                src_ref=x_ref.at[pl.ds(0, mc), :],
                dst_ref=rs_ref.at[pl.ds(i * mc, mc), :],
                send_sem=send1.at[i],
                recv_sem=recv1.at[i],
                device_id=(i,),
                device_id_type=pl.DeviceIdType.MESH,
            )
            @pl.when(my != i)
            def _(rdma=recv):
                rdma.wait_recv()

        for j in range(W):
            @pl.when(my != j)
            def _(rdma=sends1[j]):
                rdma.wait_send()

        total = jnp.sum(rs_ref[...].reshape(W, mc, n), axis=0)
        out_ref[pl.ds(my * mc, mc), :] = total

        sends2 = []
        for j in range(W):
            rdma = pltpu.make_async_remote_copy(
                src_ref=out_ref.at[pl.ds(my * mc, mc), :],
                dst_ref=out_ref.at[pl.ds(my * mc, mc), :],
                send_sem=send2.at[j],
                recv_sem=recv2.at[my],
                device_id=(j,),
                device_id_type=pl.DeviceIdType.MESH,
            )
            sends2.append(rdma)

        for j in range(W):
            @pl.when(my != j)
            def _(rdma=sends2[j]):
                rdma.start()

        for i in range(W):
            recv = pltpu.make_async_remote_copy(
                src_ref=out_ref.at[pl.ds(0, mc), :],
                dst_ref=out_ref.at[pl.ds(i * mc, mc), :],
                send_sem=send2.at[i],
                recv_sem=recv2.at[i],
                device_id=(i,),
                device_id_type=pl.DeviceIdType.MESH,
            )
            @pl.when(my != i)
            def _(rdma=recv):
                rdma.wait_recv()

        for j in range(W):
            @pl.when(my != j)
            def _(rdma=sends2[j]):
                rdma.wait_send()

    return pl.pallas_call(
        body,
        out_shape=jax.ShapeDtypeStruct((m, n), x.dtype),
        in_specs=[pl.BlockSpec(memory_space=pltpu.VMEM)],
        out_specs=pl.BlockSpec(memory_space=pltpu.VMEM),
        scratch_shapes=[
            pltpu.VMEM((m, n), x.dtype),
            pltpu.SemaphoreType.DMA((W,)),
            pltpu.SemaphoreType.DMA((W,)),
            pltpu.SemaphoreType.DMA((W,)),
            pltpu.SemaphoreType.DMA((W,)),
        ],
    )(x)


# device time: 67737 ns/iter; 1.0343x vs baseline; 1.0343x over previous
import jax
import jax.numpy as jnp
from jax import lax
from jax.experimental import pallas as pl
from jax.experimental.pallas import tpu as pltpu

W = 32


def kernel(x):
    m, n = x.shape
    mc = m // W

    def body(x_ref, out_ref, rs_ref, send1, recv1, send2, recv2):
        my = lax.axis_index("i")

        sends1 = []
        for o in range(1, W):
            j = (my + o) % W
            rdma = pltpu.make_async_remote_copy(
                src_ref=x_ref.at[pl.ds(j * mc, mc), :],
                dst_ref=rs_ref.at[pl.ds(my * mc, mc), :],
                send_sem=send1.at[o],
                recv_sem=recv1.at[my],
                device_id=(j,),
                device_id_type=pl.DeviceIdType.MESH,
            )
            rdma.start()
            sends1.append(rdma)

        total = x_ref[pl.ds(my * mc, mc), :]
        for o in range(1, W):
            i = (my - o) % W
            recv = pltpu.make_async_remote_copy(
                src_ref=x_ref.at[pl.ds(0, mc), :],
                dst_ref=rs_ref.at[pl.ds(i * mc, mc), :],
                send_sem=send1.at[o],
                recv_sem=recv1.at[i],
                device_id=(i,),
                device_id_type=pl.DeviceIdType.MESH,
            )
            recv.wait_recv()
            total = total + rs_ref[pl.ds(i * mc, mc), :]

        out_ref[pl.ds(my * mc, mc), :] = total

        sends2 = []
        for o in range(1, W):
            j = (my + o) % W
            rdma = pltpu.make_async_remote_copy(
                src_ref=out_ref.at[pl.ds(my * mc, mc), :],
                dst_ref=out_ref.at[pl.ds(my * mc, mc), :],
                send_sem=send2.at[o],
                recv_sem=recv2.at[my],
                device_id=(j,),
                device_id_type=pl.DeviceIdType.MESH,
            )
            rdma.start()
            sends2.append(rdma)

        for o in range(1, W):
            i = (my - o) % W
            recv = pltpu.make_async_remote_copy(
                src_ref=out_ref.at[pl.ds(0, mc), :],
                dst_ref=out_ref.at[pl.ds(i * mc, mc), :],
                send_sem=send2.at[o],
                recv_sem=recv2.at[i],
                device_id=(i,),
                device_id_type=pl.DeviceIdType.MESH,
            )
            recv.wait_recv()

        for rdma in sends1:
            rdma.wait_send()
        for rdma in sends2:
            rdma.wait_send()

    return pl.pallas_call(
        body,
        out_shape=jax.ShapeDtypeStruct((m, n), x.dtype),
        in_specs=[pl.BlockSpec(memory_space=pltpu.VMEM)],
        out_specs=pl.BlockSpec(memory_space=pltpu.VMEM),
        scratch_shapes=[
            pltpu.VMEM((m, n), x.dtype),
            pltpu.SemaphoreType.DMA((W,)),
            pltpu.SemaphoreType.DMA((W,)),
            pltpu.SemaphoreType.DMA((W,)),
            pltpu.SemaphoreType.DMA((W,)),
        ],
    )(x)


# device time: 63037 ns/iter; 1.1114x vs baseline; 1.0746x over previous
import jax
import jax.numpy as jnp
from jax import lax
from jax.experimental import pallas as pl
from jax.experimental.pallas import tpu as pltpu

W = 32

def _side(p: int) -> int:
    r = p % 8
    y, b = r // 2, r % 2
    return b if y % 2 == 0 else 1 - b

SIDE = [_side(p) for p in range(W)]


def kernel(x):
    m, n = x.shape
    mc = m // W

    def body(x_ref, out_ref, rs_ref, xbuf, sbuf, xs, xr, s1, r1, s2, r2):
        my = lax.axis_index("i")
        partner = my ^ 1
        r = my % 8
        side_my = jnp.where((r // 2) % 2 == 0, r % 2, 1 - r % 2)

        sends_a = []
        for j in range(W):
            rdma = pltpu.make_async_remote_copy(
                src_ref=x_ref.at[pl.ds(j * mc, mc), :],
                dst_ref=xbuf.at[pl.ds(j * mc, mc), :],
                send_sem=xs.at[j],
                recv_sem=xr.at[j],
                device_id=(partner,),
                device_id_type=pl.DeviceIdType.MESH,
            )
            sends_a.append(rdma)

            @pl.when(side_my != SIDE[j])
            def _(rdma=rdma):
                rdma.start()

        for i in range(W):
            @pl.when((side_my != SIDE[i]) | (my == i))
            def _(i=i):
                rs_ref[pl.ds(i * mc, mc), :] = jnp.zeros((mc, n), x_ref.dtype)

        sends_b = []
        for j in range(W):
            wait_a = pltpu.make_async_remote_copy(
                src_ref=x_ref.at[pl.ds(0, mc), :],
                dst_ref=xbuf.at[pl.ds(j * mc, mc), :],
                send_sem=xs.at[j],
                recv_sem=xr.at[j],
                device_id=(j,),
                device_id_type=pl.DeviceIdType.MESH,
            )

            @pl.when(side_my == SIDE[j])
            def _(rdma=wait_a):
                rdma.wait_recv()

            @pl.when((side_my == SIDE[j]) & (my != j))
            def _(j=j):
                sbuf[pl.ds(j * mc, mc), :] = (
                    x_ref[pl.ds(j * mc, mc), :] + xbuf[pl.ds(j * mc, mc), :]
                )

            rdma = pltpu.make_async_remote_copy(
                src_ref=sbuf.at[pl.ds(j * mc, mc), :],
                dst_ref=rs_ref.at[pl.ds(my * mc, mc), :],
                send_sem=s1.at[j],
                recv_sem=r1.at[my],
                device_id=(j,),
                device_id_type=pl.DeviceIdType.MESH,
            )
            sends_b.append(rdma)

            @pl.when((side_my == SIDE[j]) & (my != j))
            def _(rdma=rdma):
                rdma.start()

        total = (
            x_ref[pl.ds(my * mc, mc), :] + xbuf[pl.ds(my * mc, mc), :]
        )
        for i in range(W):
            wait_b = pltpu.make_async_remote_copy(
                src_ref=x_ref.at[pl.ds(0, mc), :],
                dst_ref=rs_ref.at[pl.ds(i * mc, mc), :],
                send_sem=s1.at[i],
                recv_sem=r1.at[i],
                device_id=(i,),
                device_id_type=pl.DeviceIdType.MESH,
            )

            @pl.when((side_my == SIDE[i]) & (my != i))
            def _(rdma=wait_b):
                rdma.wait_recv()

            total = total + rs_ref[pl.ds(i * mc, mc), :]

        out_ref[pl.ds(my * mc, mc), :] = total

        sends2 = []
        for j in range(W):
            rdma = pltpu.make_async_remote_copy(
                src_ref=out_ref.at[pl.ds(my * mc, mc), :],
                dst_ref=out_ref.at[pl.ds(my * mc, mc), :],
                send_sem=s2.at[j],
                recv_sem=r2.at[my],
                device_id=(j,),
                device_id_type=pl.DeviceIdType.MESH,
            )
            sends2.append(rdma)

            @pl.when(my != j)
            def _(rdma=rdma):
                rdma.start()

        for i in range(W):
            recv = pltpu.make_async_remote_copy(
                src_ref=out_ref.at[pl.ds(0, mc), :],
                dst_ref=out_ref.at[pl.ds(i * mc, mc), :],
                send_sem=s2.at[i],
                recv_sem=r2.at[i],
                device_id=(i,),
                device_id_type=pl.DeviceIdType.MESH,
            )

            @pl.when(my != i)
            def _(rdma=recv):
                rdma.wait_recv()

        for j in range(W):
            @pl.when(side_my != SIDE[j])
            def _(rdma=sends_a[j]):
                rdma.wait_send()

            @pl.when((side_my == SIDE[j]) & (my != j))
            def _(rdma=sends_b[j]):
                rdma.wait_send()

            @pl.when(my != j)
            def _(rdma=sends2[j]):
                rdma.wait_send()

    return pl.pallas_call(
        body,
        out_shape=jax.ShapeDtypeStruct((m, n), x.dtype),
        in_specs=[pl.BlockSpec(memory_space=pltpu.VMEM)],
        out_specs=pl.BlockSpec(memory_space=pltpu.VMEM),
        scratch_shapes=[
            pltpu.VMEM((m, n), x.dtype),
            pltpu.VMEM((m, n), x.dtype),
            pltpu.VMEM((m, n), x.dtype),
            pltpu.SemaphoreType.DMA((W,)),
            pltpu.SemaphoreType.DMA((W,)),
            pltpu.SemaphoreType.DMA((W,)),
            pltpu.SemaphoreType.DMA((W,)),
            pltpu.SemaphoreType.DMA((W,)),
            pltpu.SemaphoreType.DMA((W,)),
        ],
    )(x)
